# Initial kernel scaffold; baseline (speedup 1.0000x reference)
import jax
import jax.numpy as jnp
from jax.experimental import pallas as pl


def kernel(x, Wr, Wg, bg, Wu, bu):
    raise NotImplementedError("write your pallas kernel here")


# speedup vs baseline: 4.0584x; 4.0584x over previous
import jax
import jax.numpy as jnp
from jax.experimental import pallas as pl

_N = 8
_I = 128
_TB = 512


def _moe_body(x_ref, wrt_ref, wgt_ref, bg_ref, wu_ref, bu_ref, o_ref):
    xb = x_ref[...]
    tb = xb.shape[0]

    logits = jax.lax.dot(xb, wrt_ref[...], preferred_element_type=jnp.float32)
    n_iota = jax.lax.broadcasted_iota(jnp.int32, (tb, _N), 1)
    m1 = jnp.max(logits, axis=1, keepdims=True)
    i1 = jnp.min(jnp.where(logits == m1, n_iota, _N), axis=1, keepdims=True)
    masked = jnp.where(n_iota == i1, -jnp.inf, logits)
    m2 = jnp.max(masked, axis=1, keepdims=True)
    i2 = jnp.min(jnp.where(masked == m2, n_iota, _N), axis=1, keepdims=True)
    w1 = jax.nn.sigmoid(m1 - m2)
    comb = (jnp.where(n_iota == i1, w1, 0.0)
            + jnp.where(n_iota == i2, 1.0 - w1, 0.0))

    h = jax.lax.dot(xb.astype(jnp.bfloat16), wgt_ref[...],
                    preferred_element_type=jnp.float32)
    h = h + bg_ref[...]
    h = h * jax.nn.sigmoid(h)

    col_expert = jax.lax.broadcasted_iota(jnp.int32, (_N, _N * _I), 1) // _I
    row_expert = jax.lax.broadcasted_iota(jnp.int32, (_N, _N * _I), 0)
    expand = (col_expert == row_expert).astype(jnp.float32)
    cw = jax.lax.dot(comb, expand, preferred_element_type=jnp.float32)
    h = h * cw

    acc = jax.lax.dot(h.astype(jnp.bfloat16), wu_ref[...],
                      preferred_element_type=jnp.float32)
    acc = acc + jax.lax.dot(comb, bu_ref[...], preferred_element_type=jnp.float32)
    o_ref[...] = acc


def kernel(x, Wr, Wg, bg, Wu, bu):
    b, s, h = x.shape
    t = b * s
    xf = x.reshape(t, h)
    wrt = Wr.T
    wgt = jnp.transpose(Wg, (2, 0, 1)).reshape(h, _N * _I).astype(jnp.bfloat16)
    wu_all = jnp.transpose(Wu, (0, 2, 1)).reshape(_N * _I, h).astype(jnp.bfloat16)
    bg1 = bg.reshape(1, _N * _I)

    out = pl.pallas_call(
        _moe_body,
        grid=(t // _TB,),
        in_specs=[
            pl.BlockSpec((_TB, h), lambda i: (i, 0)),
            pl.BlockSpec((h, _N), lambda i: (0, 0)),
            pl.BlockSpec((h, _N * _I), lambda i: (0, 0)),
            pl.BlockSpec((1, _N * _I), lambda i: (0, 0)),
            pl.BlockSpec((_N * _I, h), lambda i: (0, 0)),
            pl.BlockSpec((_N, h), lambda i: (0, 0)),
        ],
        out_specs=pl.BlockSpec((_TB, h), lambda i: (i, 0)),
        out_shape=jax.ShapeDtypeStruct((t, h), jnp.float32),
    )(xf, wrt, wgt, bg1, wu_all, bu)
    return out.reshape(b, s, h)


# speedup vs baseline: 4.1175x; 1.0146x over previous
import jax
import jax.numpy as jnp
from jax.experimental import pallas as pl

_N = 8
_I = 128
_TB = 512


def _moe_body(x_ref, wr_ref, wgt_ref, bg_ref, wu_ref, eb_ref, o_ref):
    xb = x_ref[...]
    tb = xb.shape[0]

    logits_t = jax.lax.dot_general(
        wr_ref[...], xb, (((1,), (1,)), ((), ())),
        preferred_element_type=jnp.float32)
    n_iota = jax.lax.broadcasted_iota(jnp.int32, (_N, tb), 0)
    m1 = jnp.max(logits_t, axis=0, keepdims=True)
    i1 = jnp.min(jnp.where(logits_t == m1, n_iota, _N), axis=0, keepdims=True)
    masked = jnp.where(n_iota == i1, -jnp.inf, logits_t)
    m2 = jnp.max(masked, axis=0, keepdims=True)
    i2 = jnp.min(jnp.where(masked == m2, n_iota, _N), axis=0, keepdims=True)
    w1 = jax.nn.sigmoid(m1 - m2)
    comb_t = (jnp.where(n_iota == i1, w1, 0.0)
              + jnp.where(n_iota == i2, 1.0 - w1, 0.0))

    eb = jax.lax.dot_general(comb_t, eb_ref[...], (((0,), (0,)), ((), ())),
                             preferred_element_type=jnp.float32)
    cw = eb[:, :_N * _I]
    bu_term = eb[:, _N * _I:]

    h = jax.lax.dot(xb.astype(jnp.bfloat16), wgt_ref[...],
                    preferred_element_type=jnp.float32)
    h = h + bg_ref[...]
    h = h * jax.nn.sigmoid(h)
    h = h * cw

    acc = jax.lax.dot(h.astype(jnp.bfloat16), wu_ref[...],
                      preferred_element_type=jnp.float32)
    o_ref[...] = acc + bu_term


def kernel(x, Wr, Wg, bg, Wu, bu):
    b, s, h = x.shape
    t = b * s
    xf = x.reshape(t, h)
    wgt = jnp.transpose(Wg, (2, 0, 1)).reshape(h, _N * _I).astype(jnp.bfloat16)
    wu_all = jnp.transpose(Wu, (0, 2, 1)).reshape(_N * _I, h).astype(jnp.bfloat16)
    bg1 = bg.reshape(1, _N * _I)
    expand = (jnp.arange(_N * _I, dtype=jnp.int32)[None, :] // _I
              == jnp.arange(_N, dtype=jnp.int32)[:, None]).astype(jnp.float32)
    eb = jnp.concatenate([expand, bu], axis=1)

    out = pl.pallas_call(
        _moe_body,
        grid=(t // _TB,),
        in_specs=[
            pl.BlockSpec((_TB, h), lambda i: (i, 0)),
            pl.BlockSpec((_N, h), lambda i: (0, 0)),
            pl.BlockSpec((h, _N * _I), lambda i: (0, 0)),
            pl.BlockSpec((1, _N * _I), lambda i: (0, 0)),
            pl.BlockSpec((_N * _I, h), lambda i: (0, 0)),
            pl.BlockSpec((_N, _N * _I + h), lambda i: (0, 0)),
        ],
        out_specs=pl.BlockSpec((_TB, h), lambda i: (i, 0)),
        out_shape=jax.ShapeDtypeStruct((t, h), jnp.float32),
    )(xf, Wr, wgt, bg1, wu_all, eb)
    return out.reshape(b, s, h)
